# baseline (speedup 1.0000x reference)
import functools

import jax
import jax.numpy as jnp
from jax import lax
from jax.experimental import pallas as pl
from jax.experimental.pallas import tpu as pltpu
from jax.experimental.pallas import tpu_sc as plsc

NC = 2
NS = 16
L = 16


def _deg_kernel(E, N, SEG):
    nseg = E // (NC * NS) // SEG
    mesh = plsc.VectorSubcoreMesh(
        core_axis_name="c", subcore_axis_name="s", num_cores=NC, num_subcores=NS
    )
    rows_per_tile = N // NS

    @functools.partial(
        pl.kernel,
        mesh=mesh,
        out_type=jax.ShapeDtypeStruct((NC, NS, N // NS, L), jnp.float32),
        compiler_params=pltpu.CompilerParams(use_tc_tiling_on_sc=False, needs_layout_passes=False),
        scratch_types=[
            pltpu.VMEM_SHARED((N, L), jnp.float32),
            pltpu.VMEM((SEG,), jnp.int32),
            pltpu.VMEM((SEG, L), jnp.float32),
            pltpu.VMEM((rows_per_tile // 5, L), jnp.float32),
        ],
    )
    def k(col_hbm, dparts, acc, colbuf, ones, zrows):
        c = lax.axis_index("c")
        s = lax.axis_index("s")
        w = c * NS + s

        def fill_ones(i, _):
            ones[i, :] = jnp.ones((L,), jnp.float32)
            return 0

        lax.fori_loop(0, SEG, fill_ones, 0)

        zr = rows_per_tile // 5

        def fill_zero(i, _):
            zrows[i, :] = jnp.zeros((L,), jnp.float32)
            return 0

        lax.fori_loop(0, zr, fill_zero, 0)

        for i in range(5):
            pltpu.sync_copy(zrows, acc.at[pl.ds(s * rows_per_tile + i * zr, zr)])
        plsc.subcore_barrier()

        ept = E // (NC * NS)
        for g in range(nseg):
            base = w * ept + g * SEG
            pltpu.sync_copy(col_hbm.at[pl.ds(base, SEG)], colbuf)
            pltpu.sync_copy(ones, acc.at[colbuf], add=True)
        plsc.subcore_barrier()

        pltpu.sync_copy(acc.at[pl.ds(s * rows_per_tile, rows_per_tile)],
                        dparts.at[c, s])

    return k


def _scale_kernel(N, D, RB):
    def body(dp_ref, x_ref, dis_ref, y_ref):
        deg = dp_ref[0, :, 0] + dp_ref[1, :, 0] + 1.0
        dis = lax.rsqrt(deg)
        dis_ref[:, 0] = dis
        y_ref[...] = x_ref[...] * dis[:, None]

    grid = (N // RB,)
    return pl.pallas_call(
        body,
        grid=grid,
        in_specs=[
            pl.BlockSpec((NC, RB, L), lambda i: (0, i, 0)),
            pl.BlockSpec((RB, D), lambda i: (i, 0)),
        ],
        out_specs=[
            pl.BlockSpec((RB, 1), lambda i: (i, 0)),
            pl.BlockSpec((RB, D), lambda i: (i, 0)),
        ],
        out_shape=[
            jax.ShapeDtypeStruct((N, 1), jnp.float32),
            jax.ShapeDtypeStruct((N, D), jnp.float32),
        ],
    )


def _main_kernel(E, N, D, SEG, G):
    HALF = N // NC
    ACC = HALF + 8
    TRASH = HALF
    ept = E // NS
    nseg = ept // SEG
    rpt = ACC // NS
    mesh = plsc.VectorSubcoreMesh(
        core_axis_name="c", subcore_axis_name="s", num_cores=NC, num_subcores=NS
    )

    @functools.partial(
        pl.kernel,
        mesh=mesh,
        out_type=jax.ShapeDtypeStruct((NC, ACC, D), jnp.float32),
        compiler_params=pltpu.CompilerParams(use_tc_tiling_on_sc=False, needs_layout_passes=False),
        scratch_types=[
            pltpu.VMEM_SHARED((ACC, D), jnp.float32),
            pltpu.VMEM((N,), jnp.float32),
            pltpu.VMEM((SEG,), jnp.int32),
            pltpu.VMEM((SEG,), jnp.int32),
            pltpu.VMEM((SEG + G,), jnp.int32),
            pltpu.VMEM((SEG + G,), jnp.int32),
            pltpu.VMEM((SEG + G,), jnp.int32),
            pltpu.VMEM((G,), jnp.int32),
            pltpu.VMEM((G,), jnp.int32),
            pltpu.VMEM((G,), jnp.int32),
            pltpu.VMEM((G, D), jnp.float32),
            pltpu.VMEM((G, D), jnp.float32),
            pltpu.VMEM((16, D), jnp.float32),
            pltpu.SemaphoreType.DMA,
            pltpu.SemaphoreType.DMA,
        ],
    )
    def k(row_hbm, col_hbm, ep, y, dis, pre_out, acc, disbuf, rowbuf, colbuf,
          mrow, mcol, meid, gyidx, geidx, sidx, bufY, bufE, zbuf,
          semY, semE):
        c = lax.axis_index("c")
        s = lax.axis_index("s")
        lo = c * HALF

        def zb(i, _):
            for j in range(D // L):
                zbuf[i, pl.ds(j * L, L)] = jnp.zeros((L,), jnp.float32)
            return 0

        lax.fori_loop(0, 16, zb, 0)

        base = s * rpt
        nfull, rem = rpt // 16, rpt % 16

        def zcp(i, _):
            pltpu.sync_copy(zbuf, acc.at[pl.ds(base + i * 16, 16)])
            return 0

        lax.fori_loop(0, nfull, zcp, 0)
        if rem:
            pltpu.sync_copy(zbuf.at[pl.ds(0, rem)],
                            acc.at[pl.ds(base + nfull * 16, rem)])
        pltpu.sync_copy(dis, disbuf)
        plsc.subcore_barrier()

        lane = lax.iota(jnp.int32, L)
        dnums = lax.GatherDimensionNumbers(
            offset_dims=(), collapsed_slice_dims=(0,), start_index_map=(0,))

        def seg_body(g, _):
            ebase = s * ept + g * SEG
            pltpu.sync_copy(row_hbm.at[pl.ds(ebase, SEG)], rowbuf)
            pltpu.sync_copy(col_hbm.at[pl.ds(ebase, SEG)], colbuf)

            def comp(j, off):
                col16 = colbuf[pl.ds(j * L, L)] - lo
                row16 = rowbuf[pl.ds(j * L, L)]
                eid16 = (ebase + j * L) + lane
                m = (col16 >= 0) & (col16 < HALF)
                v = jnp.where(m, 1, 0)
                for k in (1, 2, 4, 8):
                    sh = lax.gather(
                        v, jnp.maximum(lane - k, 0)[:, None], dnums, (1,),
                        mode=lax.GatherScatterMode.PROMISE_IN_BOUNDS)
                    v = v + jnp.where(lane >= k, sh, 0)
                pos = off + v - 1
                plsc.store_scatter(mcol, [pos], col16, mask=m)
                plsc.store_scatter(mrow, [pos], row16, mask=m)
                plsc.store_scatter(meid, [pos], eid16, mask=m)
                return pos[L - 1] + 1

            M = lax.fori_loop(0, SEG // L, comp, jnp.int32(0))
            for t in range(G // L):
                mcol[pl.ds(M + t * L, L)] = jnp.full((L,), TRASH, jnp.int32)
                mrow[pl.ds(M + t * L, L)] = jnp.zeros((L,), jnp.int32)
                meid[pl.ds(M + t * L, L)] = jnp.zeros((L,), jnp.int32)
            nch = (M + G - 1) // G

            def chunk(kk, _):
                scales = []
                for t in range(G // L):
                    row16 = mrow[pl.ds(kk * G + t * L, L)]
                    gyidx[pl.ds(t * L, L)] = row16
                    geidx[pl.ds(t * L, L)] = meid[pl.ds(kk * G + t * L, L)]
                    sidx[pl.ds(t * L, L)] = mcol[pl.ds(kk * G + t * L, L)]
                    scales.append(plsc.load_gather(disbuf, [row16]))
                cy = pltpu.async_copy(y.at[gyidx], bufY, semY)
                ce = pltpu.async_copy(ep.at[geidx], bufE, semE)
                cy.wait()
                ce.wait()
                for gg in range(G):
                    sgg = scales[gg // L][gg % L]
                    for j in range(D // L):
                        sl = pl.ds(j * L, L)
                        bufY[gg, sl] = bufY[gg, sl] + sgg * bufE[gg, sl]
                pltpu.sync_copy(bufY, acc.at[sidx], add=True)
                return 0

            lax.fori_loop(0, nch, chunk, 0)
            return 0

        lax.fori_loop(0, nseg, seg_body, 0)
        plsc.subcore_barrier()

        pltpu.sync_copy(acc.at[pl.ds(s * rpt, rpt)],
                        pre_out.at[c, pl.ds(s * rpt, rpt)])

    return k


def _final_kernel(N, D, RB):
    HALF = N // NC
    nb_half = HALF // RB

    def body(pre_ref, dis_ref, x_ref, eps_ref, w_ref, b_ref, o_ref):
        dis = dis_ref[:, 0][:, None]
        agg = dis * pre_ref[0] + (x_ref[...] + eps_ref[...]) * (dis * dis)
        o_ref[...] = (
            lax.dot_general(
                agg, w_ref[...],
                dimension_numbers=(((1,), (1,)), ((), ())),
                precision=lax.Precision.HIGHEST,
            )
            + b_ref[...]
        )

    grid = (N // RB,)
    return pl.pallas_call(
        body,
        grid=grid,
        in_specs=[
            pl.BlockSpec((1, RB, D), lambda i: (i // nb_half, i % nb_half, 0)),
            pl.BlockSpec((RB, 1), lambda i: (i, 0)),
            pl.BlockSpec((RB, D), lambda i: (i, 0)),
            pl.BlockSpec((RB, D), lambda i: (i, 0)),
            pl.BlockSpec((D, D), lambda i: (0, 0)),
            pl.BlockSpec((1, D), lambda i: (0, 0)),
        ],
        out_specs=pl.BlockSpec((RB, D), lambda i: (i, 0)),
        out_shape=jax.ShapeDtypeStruct((N, D), jnp.float32),
    )


def kernel(x, edge_index, edge_prompt, W, bias):
    N, D = x.shape
    E = edge_index.shape[1]

    row = edge_index[0]
    col = edge_index[1]
    deg_parts = _deg_kernel(E, N, SEG=1000)(col).reshape(NC, N, L)
    dis2d, y = _scale_kernel(N, D, RB=2000)(deg_parts, x)
    pre = _main_kernel(E, N, D, SEG=2000, G=32)(
        row, col, edge_prompt, y, dis2d.reshape(N)
    )
    ep_self = edge_prompt[E:]
    out = _final_kernel(N, D, RB=1000)(
        pre, dis2d, x, ep_self, W, bias.reshape(1, D)
    )
    return out


# speedup vs baseline: 1.2836x; 1.2836x over previous
import functools

import jax
import jax.numpy as jnp
from jax import lax
from jax.experimental import pallas as pl
from jax.experimental.pallas import tpu as pltpu
from jax.experimental.pallas import tpu_sc as plsc

NC = 2
NS = 16
L = 16


def _deg_kernel(E, N, SEG):
    nseg = E // (NC * NS) // SEG
    mesh = plsc.VectorSubcoreMesh(
        core_axis_name="c", subcore_axis_name="s", num_cores=NC, num_subcores=NS
    )
    rows_per_tile = N // NS

    @functools.partial(
        pl.kernel,
        mesh=mesh,
        out_type=jax.ShapeDtypeStruct((NC, NS, N // NS, L), jnp.float32),
        compiler_params=pltpu.CompilerParams(use_tc_tiling_on_sc=False, needs_layout_passes=False),
        scratch_types=[
            pltpu.VMEM_SHARED((N, L), jnp.float32),
            pltpu.VMEM((SEG,), jnp.int32),
            pltpu.VMEM((SEG, L), jnp.float32),
            pltpu.VMEM((rows_per_tile // 5, L), jnp.float32),
        ],
    )
    def k(col_hbm, dparts, acc, colbuf, ones, zrows):
        c = lax.axis_index("c")
        s = lax.axis_index("s")
        w = c * NS + s

        def fill_ones(i, _):
            ones[i, :] = jnp.ones((L,), jnp.float32)
            return 0

        lax.fori_loop(0, SEG, fill_ones, 0)

        zr = rows_per_tile // 5

        def fill_zero(i, _):
            zrows[i, :] = jnp.zeros((L,), jnp.float32)
            return 0

        lax.fori_loop(0, zr, fill_zero, 0)

        for i in range(5):
            pltpu.sync_copy(zrows, acc.at[pl.ds(s * rows_per_tile + i * zr, zr)])
        plsc.subcore_barrier()

        ept = E // (NC * NS)
        for g in range(nseg):
            base = w * ept + g * SEG
            pltpu.sync_copy(col_hbm.at[pl.ds(base, SEG)], colbuf)
            pltpu.sync_copy(ones, acc.at[colbuf], add=True)
        plsc.subcore_barrier()

        pltpu.sync_copy(acc.at[pl.ds(s * rows_per_tile, rows_per_tile)],
                        dparts.at[c, s])

    return k


def _scale_kernel(N, D, RB):
    def body(dp_ref, x_ref, dis_ref, y_ref):
        deg = dp_ref[0, :, 0] + dp_ref[1, :, 0] + 1.0
        dis = lax.rsqrt(deg)
        dis_ref[:, 0] = dis
        y_ref[...] = x_ref[...] * dis[:, None]

    grid = (N // RB,)
    return pl.pallas_call(
        body,
        grid=grid,
        in_specs=[
            pl.BlockSpec((NC, RB, L), lambda i: (0, i, 0)),
            pl.BlockSpec((RB, D), lambda i: (i, 0)),
        ],
        out_specs=[
            pl.BlockSpec((RB, 1), lambda i: (i, 0)),
            pl.BlockSpec((RB, D), lambda i: (i, 0)),
        ],
        out_shape=[
            jax.ShapeDtypeStruct((N, 1), jnp.float32),
            jax.ShapeDtypeStruct((N, D), jnp.float32),
        ],
    )


def _main_kernel(E, N, D, SEG, G):
    HALF = N // NC
    ACC = HALF + 8
    TRASH = HALF
    ept = E // NS
    nseg = ept // SEG
    rpt = ACC // NS
    mesh = plsc.VectorSubcoreMesh(
        core_axis_name="c", subcore_axis_name="s", num_cores=NC, num_subcores=NS
    )

    @functools.partial(
        pl.kernel,
        mesh=mesh,
        out_type=jax.ShapeDtypeStruct((NC, ACC, D), jnp.float32),
        compiler_params=pltpu.CompilerParams(use_tc_tiling_on_sc=False, needs_layout_passes=False),
        scratch_types=[
            pltpu.VMEM_SHARED((ACC, D), jnp.float32),
            pltpu.VMEM((N,), jnp.float32),
            pltpu.VMEM((SEG,), jnp.int32),
            pltpu.VMEM((SEG,), jnp.int32),
            pltpu.VMEM((SEG + G,), jnp.int32),
            pltpu.VMEM((SEG + G,), jnp.int32),
            pltpu.VMEM((SEG + G,), jnp.int32),
            pltpu.VMEM((2, G), jnp.int32),
            pltpu.VMEM((2, G), jnp.int32),
            pltpu.VMEM((2, G), jnp.int32),
            pltpu.VMEM((2, G, D), jnp.float32),
            pltpu.VMEM((2, G, D), jnp.float32),
            pltpu.VMEM((16, D), jnp.float32),
            pltpu.SemaphoreType.DMA,
            pltpu.SemaphoreType.DMA,
            pltpu.SemaphoreType.DMA,
            pltpu.SemaphoreType.DMA,
            pltpu.SemaphoreType.DMA,
            pltpu.SemaphoreType.DMA,
        ],
    )
    def k(row_hbm, col_hbm, ep, y, dis, pre_out, acc, disbuf, rowbuf, colbuf,
          mrow, mcol, meid, gyidx, geidx, sidx, bufY, bufE, zbuf,
          semY0, semY1, semE0, semE1, semS0, semS1):
        c = lax.axis_index("c")
        s = lax.axis_index("s")
        lo = c * HALF

        def zb(i, _):
            for j in range(D // L):
                zbuf[i, pl.ds(j * L, L)] = jnp.zeros((L,), jnp.float32)
            return 0

        lax.fori_loop(0, 16, zb, 0)

        base = s * rpt
        nfull, rem = rpt // 16, rpt % 16

        def zcp(i, _):
            pltpu.sync_copy(zbuf, acc.at[pl.ds(base + i * 16, 16)])
            return 0

        lax.fori_loop(0, nfull, zcp, 0)
        if rem:
            pltpu.sync_copy(zbuf.at[pl.ds(0, rem)],
                            acc.at[pl.ds(base + nfull * 16, rem)])
        pltpu.sync_copy(dis, disbuf)
        plsc.subcore_barrier()

        lane = lax.iota(jnp.int32, L)
        dnums = lax.GatherDimensionNumbers(
            offset_dims=(), collapsed_slice_dims=(0,), start_index_map=(0,))

        def seg_body(g, _):
            ebase = s * ept + g * SEG
            pltpu.sync_copy(row_hbm.at[pl.ds(ebase, SEG)], rowbuf)
            pltpu.sync_copy(col_hbm.at[pl.ds(ebase, SEG)], colbuf)

            def comp(j, off):
                col16 = colbuf[pl.ds(j * L, L)] - lo
                row16 = rowbuf[pl.ds(j * L, L)]
                eid16 = (ebase + j * L) + lane
                m = (col16 >= 0) & (col16 < HALF)
                v = jnp.where(m, 1, 0)
                for k in (1, 2, 4, 8):
                    sh = lax.gather(
                        v, jnp.maximum(lane - k, 0)[:, None], dnums, (1,),
                        mode=lax.GatherScatterMode.PROMISE_IN_BOUNDS)
                    v = v + jnp.where(lane >= k, sh, 0)
                pos = off + v - 1
                plsc.store_scatter(mcol, [pos], col16, mask=m)
                plsc.store_scatter(mrow, [pos], row16, mask=m)
                plsc.store_scatter(meid, [pos], eid16, mask=m)
                return pos[L - 1] + 1

            M = lax.fori_loop(0, SEG // L, comp, jnp.int32(0))
            for t in range(G // L):
                mcol[pl.ds(M + t * L, L)] = jnp.full((L,), TRASH, jnp.int32)
                mrow[pl.ds(M + t * L, L)] = jnp.zeros((L,), jnp.int32)
                meid[pl.ds(M + t * L, L)] = jnp.zeros((L,), jnp.int32)
            nch = (M + G - 1) // G
            semY = (semY0, semY1)
            semE = (semE0, semE1)
            semS = (semS0, semS1)

            def issue(b, kk):
                gyidx.at[b][...] = mrow[pl.ds(kk * G, G)]
                geidx.at[b][...] = meid[pl.ds(kk * G, G)]
                sidx.at[b][...] = mcol[pl.ds(kk * G, G)]
                pltpu.async_copy(y.at[gyidx.at[b]], bufY.at[b], semY[b])
                pltpu.async_copy(ep.at[geidx.at[b]], bufE.at[b], semE[b])

            def drain_scatter(b):
                pltpu.make_async_copy(bufY.at[b], acc.at[sidx.at[b]],
                                      semS[b]).wait()

            def step(kk, b):
                nb = 1 - b

                @pl.when(kk + 1 < nch)
                def _():
                    @pl.when(kk + 1 >= 2)
                    def _():
                        drain_scatter(nb)
                    issue(nb, kk + 1)

                pltpu.make_async_copy(y.at[gyidx.at[b]], bufY.at[b],
                                      semY[b]).wait()
                pltpu.make_async_copy(ep.at[geidx.at[b]], bufE.at[b],
                                      semE[b]).wait()
                sc16 = plsc.load_gather(disbuf, [mrow[pl.ds(kk * G, L)]])
                for gg in range(G):
                    sgg = sc16[gg]
                    for j in range(D // L):
                        sl = pl.ds(j * L, L)
                        bufY.at[b][gg, sl] = (bufY.at[b][gg, sl]
                                              + sgg * bufE.at[b][gg, sl])
                pltpu.async_copy(bufY.at[b], acc.at[sidx.at[b]], semS[b],
                                 add=True)

            @pl.when(nch > 0)
            def _():
                issue(0, 0)

            def pair(p, _):
                @pl.when(2 * p < nch)
                def _():
                    step(2 * p, 0)

                @pl.when(2 * p + 1 < nch)
                def _():
                    step(2 * p + 1, 1)

                return 0

            lax.fori_loop(0, (nch + 1) // 2, pair, 0)

            @pl.when(nch >= 1)
            def _():
                drain_scatter(0)

            @pl.when(nch >= 2)
            def _():
                drain_scatter(1)
            return 0

        lax.fori_loop(0, nseg, seg_body, 0)
        plsc.subcore_barrier()

        pltpu.sync_copy(acc.at[pl.ds(s * rpt, rpt)],
                        pre_out.at[c, pl.ds(s * rpt, rpt)])

    return k


def _final_kernel(N, D, RB):
    HALF = N // NC
    nb_half = HALF // RB

    def body(pre_ref, dis_ref, x_ref, eps_ref, w_ref, b_ref, o_ref):
        dis = dis_ref[:, 0][:, None]
        agg = dis * pre_ref[0] + (x_ref[...] + eps_ref[...]) * (dis * dis)
        o_ref[...] = (
            lax.dot_general(
                agg, w_ref[...],
                dimension_numbers=(((1,), (1,)), ((), ())),
                precision=lax.Precision.HIGHEST,
            )
            + b_ref[...]
        )

    grid = (N // RB,)
    return pl.pallas_call(
        body,
        grid=grid,
        in_specs=[
            pl.BlockSpec((1, RB, D), lambda i: (i // nb_half, i % nb_half, 0)),
            pl.BlockSpec((RB, 1), lambda i: (i, 0)),
            pl.BlockSpec((RB, D), lambda i: (i, 0)),
            pl.BlockSpec((RB, D), lambda i: (i, 0)),
            pl.BlockSpec((D, D), lambda i: (0, 0)),
            pl.BlockSpec((1, D), lambda i: (0, 0)),
        ],
        out_specs=pl.BlockSpec((RB, D), lambda i: (i, 0)),
        out_shape=jax.ShapeDtypeStruct((N, D), jnp.float32),
    )


def kernel(x, edge_index, edge_prompt, W, bias):
    N, D = x.shape
    E = edge_index.shape[1]

    row = edge_index[0]
    col = edge_index[1]
    deg_parts = _deg_kernel(E, N, SEG=1000)(col).reshape(NC, N, L)
    dis2d, y = _scale_kernel(N, D, RB=2000)(deg_parts, x)
    pre = _main_kernel(E, N, D, SEG=2000, G=16)(
        row, col, edge_prompt, y, dis2d.reshape(N)
    )
    ep_self = edge_prompt[E:]
    out = _final_kernel(N, D, RB=1000)(
        pre, dis2d, x, ep_self, W, bias.reshape(1, D)
    )
    return out
